# baseline (speedup 1.0000x reference)
import functools

import jax
import jax.numpy as jnp
from jax import lax
from jax.experimental import pallas as pl
from jax.experimental.pallas import tpu as pltpu
from jax.experimental.pallas import tpu_sc as plsc

N_FEATURES = 26
VOCAB = 100000
EMBED_DIM = 32
BATCH = 16384
IN_DIM = N_FEATURES * EMBED_DIM
H0, H1 = 512, 256

NC, NS = 2, 16
NW = NC * NS
RPW = IN_DIM // NW
OUT_CH = 4096
N_CH = BATCH // OUT_CH


def _sc_gather_t(tabT, idxT, tabTail):
    mesh = plsc.VectorSubcoreMesh(core_axis_name="c", subcore_axis_name="s")

    @functools.partial(
        pl.kernel,
        out_type=jax.ShapeDtypeStruct((IN_DIM, BATCH), jnp.float32),
        mesh=mesh,
        scratch_types=[
            pltpu.VMEM((VOCAB,), jnp.float32),
            pltpu.VMEM((BATCH,), jnp.int32),
            pltpu.VMEM((OUT_CH,), jnp.float32),
            pltpu.VMEM((OUT_CH,), jnp.float32),
            pltpu.SemaphoreType.DMA,
            pltpu.SemaphoreType.DMA,
            pltpu.SemaphoreType.DMA,
            pltpu.SemaphoreType.DMA,
            pltpu.SemaphoreType.DMA,
            pltpu.SemaphoreType.DMA,
        ],
        compiler_params=pltpu.CompilerParams(
            needs_layout_passes=False, use_tc_tiling_on_sc=True),
    )
    def k(tab_hbm, idx_hbm, tail_hbm, out_hbm, rowbuf, idxbuf, ob0, ob1,
          s0, s1, r0s, r1s, r2s, r3s):
        wid = lax.axis_index("s") * NC + lax.axis_index("c")
        r0 = wid * RPW
        obufs = (ob0, ob1)
        sems = (s0, s1)

        def row_body(i, _):
            r = r0 + i
            f = r // EMBED_DIM

            @pl.when(jnp.logical_or(i == 0, r % EMBED_DIM == 0))
            def _load_idx():
                pltpu.sync_copy(idx_hbm.at[f], idxbuf)

            rsems = (r0s, r1s, r2s, r3s)
            bounds = (0, 25088, 50176, 75264, 99968, VOCAB)
            descs = []
            for q in range(4):
                lo, w = bounds[q], bounds[q + 1] - bounds[q]
                descs.append(pltpu.async_copy(
                    tab_hbm.at[r, pl.ds(lo, w)], rowbuf.at[pl.ds(lo, w)],
                    rsems[q]))
            pltpu.sync_copy(tail_hbm.at[pl.ds(r * 32, 32)],
                            rowbuf.at[pl.ds(99968, 32)])
            for d in descs:
                d.wait()

            for h in range(N_CH):
                ob, sem = obufs[h % 2], sems[h % 2]

                def _drain(ob=ob, sem=sem, h=h):
                    pltpu.make_async_copy(
                        ob, out_hbm.at[r, pl.ds(h * OUT_CH, OUT_CH)], sem).wait()

                if h < 2:
                    pl.when(i > 0)(_drain)
                else:
                    _drain()

                @plsc.parallel_loop(0, OUT_CH, step=16, unroll=8)
                def _vec(j):
                    iv = idxbuf[pl.ds(h * OUT_CH + j, 16)]
                    ob[pl.ds(j, 16)] = plsc.load_gather(rowbuf, [iv])

                pltpu.async_copy(
                    ob, out_hbm.at[r, pl.ds(h * OUT_CH, OUT_CH)], sem)
            return 0

        lax.fori_loop(0, RPW, row_body, 0)
        for h in range(2):
            pltpu.make_async_copy(
                obufs[h], out_hbm.at[r0, pl.ds(h * OUT_CH, OUT_CH)],
                sems[h]).wait()

    return k(tabT, idxT, tabTail)


BM = 1024


def _mlp_body(xT_ref, w0_ref, b0_ref, w1_ref, b1_ref, w2_ref, b2_ref, o_ref):
    xT = xT_ref[...]
    h = lax.dot_general(xT, w0_ref[...], (((0,), (0,)), ((), ())),
                        preferred_element_type=jnp.float32) + b0_ref[...]
    h = h * jax.nn.sigmoid(h)
    h = jnp.dot(h, w1_ref[...], preferred_element_type=jnp.float32) + b1_ref[...]
    h = h * jax.nn.sigmoid(h)
    h = jnp.dot(h, w2_ref[...], preferred_element_type=jnp.float32) + b2_ref[...]
    norm = jnp.sqrt(jnp.sum(h * h, axis=-1, keepdims=True))
    o_ref[...] = h / jnp.maximum(norm, 1e-12)


def _mlp(xT, W0, b0, W1, b1, W2, b2):
    grid = (BATCH // BM,)
    return pl.pallas_call(
        _mlp_body,
        grid=grid,
        in_specs=[
            pl.BlockSpec((IN_DIM, BM), lambda i: (0, i)),
            pl.BlockSpec((IN_DIM, H0), lambda i: (0, 0)),
            pl.BlockSpec((1, H0), lambda i: (0, 0)),
            pl.BlockSpec((H0, H1), lambda i: (0, 0)),
            pl.BlockSpec((1, H1), lambda i: (0, 0)),
            pl.BlockSpec((H1, EMBED_DIM), lambda i: (0, 0)),
            pl.BlockSpec((1, EMBED_DIM), lambda i: (0, 0)),
        ],
        out_specs=pl.BlockSpec((BM, EMBED_DIM), lambda i: (i, 0)),
        out_shape=jax.ShapeDtypeStruct((BATCH, EMBED_DIM), jnp.float32),
        compiler_params=pltpu.CompilerParams(
            dimension_semantics=("arbitrary",),
        ),
    )(xT, W0, b0, W1, b1, W2, b2)


def kernel(indices, tables, W0, b0, W1, b1, W2, b2):
    tabT = jnp.transpose(tables, (0, 2, 1)).reshape(IN_DIM, VOCAB)
    idxT = indices.astype(jnp.int32).T
    tabTail = tabT[:, 99968:].reshape(-1)
    xT = _sc_gather_t(tabT, idxT, tabTail)
    return _mlp(xT, W0, b0.reshape(1, H0), W1, b1.reshape(1, H1),
                W2, b2.reshape(1, EMBED_DIM))


# speedup vs baseline: 1.0522x; 1.0522x over previous
import functools

import jax
import jax.numpy as jnp
from jax import lax
from jax.experimental import pallas as pl
from jax.experimental.pallas import tpu as pltpu
from jax.experimental.pallas import tpu_sc as plsc

N_FEATURES = 26
VOCAB = 100000
EMBED_DIM = 32
BATCH = 16384
IN_DIM = N_FEATURES * EMBED_DIM
H0, H1 = 512, 256

NC, NS = 2, 16
NW = NC * NS
RPW = IN_DIM // NW
OUT_CH = 4096
N_CH = BATCH // OUT_CH


def _sc_gather_t(tabT, idxT):
    mesh = plsc.VectorSubcoreMesh(core_axis_name="c", subcore_axis_name="s")

    @functools.partial(
        pl.kernel,
        out_type=jax.ShapeDtypeStruct((IN_DIM, BATCH), jnp.float32),
        mesh=mesh,
        scratch_types=[
            pltpu.VMEM((VOCAB,), jnp.float32),
            pltpu.VMEM((BATCH,), jnp.int32),
            pltpu.VMEM((OUT_CH,), jnp.float32),
            pltpu.VMEM((OUT_CH,), jnp.float32),
            pltpu.SemaphoreType.DMA,
            pltpu.SemaphoreType.DMA,
        ],
        compiler_params=pltpu.CompilerParams(
            needs_layout_passes=False, use_tc_tiling_on_sc=True),
    )
    def k(tab_hbm, idx_hbm, out_hbm, rowbuf, idxbuf, ob0, ob1, s0, s1):
        wid = lax.axis_index("s") * NC + lax.axis_index("c")
        r0 = wid * RPW
        obufs = (ob0, ob1)
        sems = (s0, s1)

        def row_body(i, _):
            r = r0 + i
            f = r // EMBED_DIM

            @pl.when(jnp.logical_or(i == 0, r % EMBED_DIM == 0))
            def _load_idx():
                pltpu.sync_copy(idx_hbm.at[f], idxbuf)

            pltpu.sync_copy(tab_hbm.at[r], rowbuf)

            for h in range(N_CH):
                ob, sem = obufs[h % 2], sems[h % 2]

                def _drain(ob=ob, sem=sem, h=h):
                    pltpu.make_async_copy(
                        ob, out_hbm.at[r, pl.ds(h * OUT_CH, OUT_CH)], sem).wait()

                if h < 2:
                    pl.when(i > 0)(_drain)
                else:
                    _drain()

                @plsc.parallel_loop(0, OUT_CH, step=16, unroll=8)
                def _vec(j):
                    iv = idxbuf[pl.ds(h * OUT_CH + j, 16)]
                    ob[pl.ds(j, 16)] = plsc.load_gather(rowbuf, [iv])

                pltpu.async_copy(
                    ob, out_hbm.at[r, pl.ds(h * OUT_CH, OUT_CH)], sem)
            return 0

        lax.fori_loop(0, RPW, row_body, 0)
        for h in range(2):
            pltpu.make_async_copy(
                obufs[h], out_hbm.at[r0, pl.ds(h * OUT_CH, OUT_CH)],
                sems[h]).wait()

    return k(tabT, idxT)


BM = 2048


def _mlp_body(xT_ref, w0_ref, b0_ref, w1_ref, b1_ref, w2_ref, b2_ref, o_ref):
    xT = xT_ref[...].astype(jnp.bfloat16)
    h = lax.dot_general(xT, w0_ref[...].astype(jnp.bfloat16),
                        (((0,), (0,)), ((), ())),
                        preferred_element_type=jnp.float32) + b0_ref[...]
    h = h * jax.nn.sigmoid(h)
    h = jnp.dot(h.astype(jnp.bfloat16), w1_ref[...].astype(jnp.bfloat16),
                preferred_element_type=jnp.float32) + b1_ref[...]
    h = h * jax.nn.sigmoid(h)
    h = jnp.dot(h.astype(jnp.bfloat16), w2_ref[...].astype(jnp.bfloat16),
                preferred_element_type=jnp.float32) + b2_ref[...]
    norm = jnp.sqrt(jnp.sum(h * h, axis=-1, keepdims=True))
    o_ref[...] = h / jnp.maximum(norm, 1e-12)


def _mlp(xT, W0, b0, W1, b1, W2, b2):
    grid = (BATCH // BM,)
    return pl.pallas_call(
        _mlp_body,
        grid=grid,
        in_specs=[
            pl.BlockSpec((IN_DIM, BM), lambda i: (0, i)),
            pl.BlockSpec((IN_DIM, H0), lambda i: (0, 0)),
            pl.BlockSpec((1, H0), lambda i: (0, 0)),
            pl.BlockSpec((H0, H1), lambda i: (0, 0)),
            pl.BlockSpec((1, H1), lambda i: (0, 0)),
            pl.BlockSpec((H1, EMBED_DIM), lambda i: (0, 0)),
            pl.BlockSpec((1, EMBED_DIM), lambda i: (0, 0)),
        ],
        out_specs=pl.BlockSpec((BM, EMBED_DIM), lambda i: (i, 0)),
        out_shape=jax.ShapeDtypeStruct((BATCH, EMBED_DIM), jnp.float32),
        compiler_params=pltpu.CompilerParams(
            dimension_semantics=("arbitrary",),
            vmem_limit_bytes=100 * 1024 * 1024,
        ),
    )(xT, W0, b0, W1, b1, W2, b2)


def kernel(indices, tables, W0, b0, W1, b1, W2, b2):
    tabT = jnp.transpose(tables, (0, 2, 1)).reshape(IN_DIM, VOCAB)
    idxT = indices.astype(jnp.int32).T
    xT = _sc_gather_t(tabT, idxT)
    return _mlp(xT, W0, b0.reshape(1, H0), W1, b1.reshape(1, H1),
                W2, b2.reshape(1, EMBED_DIM))


# speedup vs baseline: 1.0575x; 1.0050x over previous
import functools

import jax
import jax.numpy as jnp
from jax import lax
from jax.experimental import pallas as pl
from jax.experimental.pallas import tpu as pltpu
from jax.experimental.pallas import tpu_sc as plsc

N_FEATURES = 26
VOCAB = 100000
EMBED_DIM = 32
BATCH = 16384
IN_DIM = N_FEATURES * EMBED_DIM
H0, H1 = 512, 256

NC, NS = 2, 16
NW = NC * NS
RPW = IN_DIM // NW
OUT_CH = 4096
N_CH = BATCH // OUT_CH


def _sc_gather_t(tabT, idxT):
    mesh = plsc.VectorSubcoreMesh(core_axis_name="c", subcore_axis_name="s")

    @functools.partial(
        pl.kernel,
        out_type=jax.ShapeDtypeStruct((IN_DIM, BATCH), jnp.float32),
        mesh=mesh,
        scratch_types=[
            pltpu.VMEM((VOCAB,), jnp.float32),
            pltpu.VMEM((BATCH,), jnp.int32),
            pltpu.VMEM((OUT_CH,), jnp.float32),
            pltpu.VMEM((OUT_CH,), jnp.float32),
            pltpu.SemaphoreType.DMA,
            pltpu.SemaphoreType.DMA,
        ],
        compiler_params=pltpu.CompilerParams(
            needs_layout_passes=False, use_tc_tiling_on_sc=True),
    )
    def k(tab_hbm, idx_hbm, out_hbm, rowbuf, idxbuf, ob0, ob1, s0, s1):
        wid = lax.axis_index("s") * NC + lax.axis_index("c")
        r0 = wid * RPW
        obufs = (ob0, ob1)
        sems = (s0, s1)

        def row_body(i, _):
            r = r0 + i
            f = r // EMBED_DIM

            @pl.when(jnp.logical_or(i == 0, r % EMBED_DIM == 0))
            def _load_idx():
                pltpu.sync_copy(idx_hbm.at[f], idxbuf)

            pltpu.sync_copy(tab_hbm.at[r], rowbuf)

            for h in range(N_CH):
                ob, sem = obufs[h % 2], sems[h % 2]

                def _drain(ob=ob, sem=sem, h=h):
                    pltpu.make_async_copy(
                        ob, out_hbm.at[r, pl.ds(h * OUT_CH, OUT_CH)], sem).wait()

                if h < 2:
                    pl.when(i > 0)(_drain)
                else:
                    _drain()

                @plsc.parallel_loop(0, OUT_CH, step=16, unroll=16)
                def _vec(j):
                    iv = idxbuf[pl.ds(h * OUT_CH + j, 16)]
                    ob[pl.ds(j, 16)] = plsc.load_gather(rowbuf, [iv])

                pltpu.async_copy(
                    ob, out_hbm.at[r, pl.ds(h * OUT_CH, OUT_CH)], sem)
            return 0

        lax.fori_loop(0, RPW, row_body, 0)
        for h in range(2):
            pltpu.make_async_copy(
                obufs[h], out_hbm.at[r0, pl.ds(h * OUT_CH, OUT_CH)],
                sems[h]).wait()

    return k(tabT, idxT)


BM = 2048


def _mlp_body(xT_ref, w0_ref, b0_ref, w1_ref, b1_ref, w2_ref, b2_ref, o_ref):
    xT = xT_ref[...].astype(jnp.bfloat16)
    h = lax.dot_general(xT, w0_ref[...].astype(jnp.bfloat16),
                        (((0,), (0,)), ((), ())),
                        preferred_element_type=jnp.float32) + b0_ref[...]
    h = h * jax.nn.sigmoid(h)
    h = jnp.dot(h.astype(jnp.bfloat16), w1_ref[...].astype(jnp.bfloat16),
                preferred_element_type=jnp.float32) + b1_ref[...]
    h = h * jax.nn.sigmoid(h)
    h = jnp.dot(h.astype(jnp.bfloat16), w2_ref[...].astype(jnp.bfloat16),
                preferred_element_type=jnp.float32) + b2_ref[...]
    norm = jnp.sqrt(jnp.sum(h * h, axis=-1, keepdims=True))
    o_ref[...] = h / jnp.maximum(norm, 1e-12)


def _mlp(xT, W0, b0, W1, b1, W2, b2):
    grid = (BATCH // BM,)
    return pl.pallas_call(
        _mlp_body,
        grid=grid,
        in_specs=[
            pl.BlockSpec((IN_DIM, BM), lambda i: (0, i)),
            pl.BlockSpec((IN_DIM, H0), lambda i: (0, 0)),
            pl.BlockSpec((1, H0), lambda i: (0, 0)),
            pl.BlockSpec((H0, H1), lambda i: (0, 0)),
            pl.BlockSpec((1, H1), lambda i: (0, 0)),
            pl.BlockSpec((H1, EMBED_DIM), lambda i: (0, 0)),
            pl.BlockSpec((1, EMBED_DIM), lambda i: (0, 0)),
        ],
        out_specs=pl.BlockSpec((BM, EMBED_DIM), lambda i: (i, 0)),
        out_shape=jax.ShapeDtypeStruct((BATCH, EMBED_DIM), jnp.float32),
        compiler_params=pltpu.CompilerParams(
            dimension_semantics=("arbitrary",),
            vmem_limit_bytes=100 * 1024 * 1024,
        ),
    )(xT, W0, b0, W1, b1, W2, b2)


def kernel(indices, tables, W0, b0, W1, b1, W2, b2):
    tabT = jnp.transpose(tables, (0, 2, 1)).reshape(IN_DIM, VOCAB)
    idxT = indices.astype(jnp.int32).T
    xT = _sc_gather_t(tabT, idxT)
    return _mlp(xT, W0, b0.reshape(1, H0), W1, b1.reshape(1, H1),
                W2, b2.reshape(1, EMBED_DIM))
